# baseline (speedup 1.0000x reference)
import functools

import jax
import jax.numpy as jnp
from jax import lax
from jax.experimental import pallas as pl
from jax.experimental.pallas import tpu as pltpu
from jax.experimental.pallas import tpu_sc as plsc

DIM = 64
B = 4096
L = 50
NB = B * L
NC = 2
NS = 16
NW = NC * NS
BPW = NB // NW
CW = 128
NCHUNK = BPW // CW

_mesh = plsc.VectorSubcoreMesh(core_axis_name="c", subcore_axis_name="s")


@functools.partial(
    pl.kernel,
    mesh=_mesh,
    out_type=jax.ShapeDtypeStruct((NB, DIM), jnp.float32),
    scratch_types=[
        pltpu.VMEM((NCHUNK, CW), jnp.int32),
        pltpu.VMEM((2, CW, DIM), jnp.float32),
        pltpu.SemaphoreType.DMA,
    ],
    compiler_params=pltpu.CompilerParams(use_tc_tiling_on_sc=False),
)
def _embed_gather(idx_hbm, table_hbm, out_hbm, idx_v, rows_v, gsem):
    wid = lax.axis_index("s") * NC + lax.axis_index("c")
    base = wid * BPW
    pltpu.sync_copy(idx_hbm.at[wid], idx_v)

    def body(i, carry):
        for b in range(2):
            j = i * 2 + b
            pltpu.async_copy(table_hbm.at[idx_v.at[j]], rows_v.at[b], gsem).wait()
            pltpu.sync_copy(rows_v.at[b], out_hbm.at[pl.ds(base + j * CW, CW)])
        return carry

    lax.fori_loop(0, NCHUNK // 2, body, 0)


def kernel(x, weight):
    idx = x.reshape(NW, NCHUNK, CW).astype(jnp.int32)
    out = _embed_gather(idx, weight)
    return out.reshape(B, L, DIM)


# speedup vs baseline: 1.0454x; 1.0454x over previous
import functools

import jax
import jax.numpy as jnp
from jax import lax
from jax.experimental import pallas as pl
from jax.experimental.pallas import tpu as pltpu
from jax.experimental.pallas import tpu_sc as plsc

DIM = 64
B = 4096
L = 50
NB = B * L
NC = 2
NS = 16
NW = NC * NS
BPW = NB // NW
CW = 128
NCHUNK = BPW // CW

_mesh = plsc.VectorSubcoreMesh(core_axis_name="c", subcore_axis_name="s")


@functools.partial(
    pl.kernel,
    mesh=_mesh,
    out_type=jax.ShapeDtypeStruct((NB, DIM), jnp.float32),
    scratch_types=[
        pltpu.VMEM((NCHUNK, CW), jnp.int32),
        pltpu.VMEM((5, CW, DIM), jnp.float32),
        pltpu.SemaphoreType.DMA,
        pltpu.SemaphoreType.DMA,
    ],
    compiler_params=pltpu.CompilerParams(use_tc_tiling_on_sc=False),
)
def _embed_gather(idx_hbm, table_hbm, out_hbm, idx_v, rows_v, gsem, osem):
    NBUF = 5
    wid = lax.axis_index("s") * NC + lax.axis_index("c")
    base = wid * BPW
    pltpu.sync_copy(idx_hbm.at[wid], idx_v)

    for b in range(NBUF):
        pltpu.async_copy(table_hbm.at[idx_v.at[b]], rows_v.at[b], gsem)

    def body(i, carry):
        for b in range(NBUF):
            j = i * NBUF + b
            pltpu.make_async_copy(
                table_hbm.at[idx_v.at[b]], rows_v.at[b], gsem
            ).wait()
            pltpu.async_copy(
                rows_v.at[b], out_hbm.at[pl.ds(base + j * CW, CW)], osem
            ).wait()
            nxt = j + NBUF

            @pl.when(nxt < NCHUNK)
            def _():
                pltpu.async_copy(table_hbm.at[idx_v.at[nxt]], rows_v.at[b], gsem)

        return carry

    lax.fori_loop(0, NCHUNK // NBUF, body, 0)


def kernel(x, weight):
    idx = x.reshape(NW, NCHUNK, CW).astype(jnp.int32)
    out = _embed_gather(idx, weight)
    return out.reshape(B, L, DIM)
